# baseline (speedup 1.0000x reference)
import functools

import jax
import jax.numpy as jnp
from jax import lax
from jax.experimental import pallas as pl
from jax.experimental.pallas import tpu as pltpu
from jax.experimental.pallas import tpu_sc as plsc

D = 128
D2 = D // 2
K = 32
NC = 2
NS = 16
NW = NC * NS
CPW = D2 // NW
L = 16
CH = 2048


def _pack_bf16_pair(lo_f32, hi_f32):
    lo = lax.convert_element_type(
        lax.bitcast_convert_type(lo_f32.astype(jnp.bfloat16), jnp.uint16),
        jnp.uint32)
    hi = lax.convert_element_type(
        lax.bitcast_convert_type(hi_f32.astype(jnp.bfloat16), jnp.uint16),
        jnp.uint32)
    return lax.bitcast_convert_type((hi << 16) | lo, jnp.int32)


def _matmul_body(x_ref, wte_ref, wto_ref, wde_ref, wdo_ref, hs_ref, hd_ref):
    x = x_ref[...]
    hse = jnp.dot(x, wte_ref[...], preferred_element_type=jnp.float32)
    hso = jnp.dot(x, wto_ref[...], preferred_element_type=jnp.float32)
    hde = jnp.dot(x, wde_ref[...], preferred_element_type=jnp.float32)
    hdo = jnp.dot(x, wdo_ref[...], preferred_element_type=jnp.float32)
    hs_ref[...] = _pack_bf16_pair(hse, hso).T
    hd_ref[...] = _pack_bf16_pair(hde, hdo)


def _tc_matmuls(feat_pad, wt, wd, n_pad):
    bm = 512
    grid = (n_pad // bm,)
    wb = [w.astype(jnp.bfloat16)
          for w in (wt[:, :D2], wt[:, D2:], wd[:, :D2], wd[:, D2:])]
    return pl.pallas_call(
        _matmul_body,
        grid=grid,
        in_specs=[pl.BlockSpec((bm, D), lambda i: (i, 0))] + [
            pl.BlockSpec((D, D2), lambda i: (0, 0))] * 4,
        out_specs=[
            pl.BlockSpec((D2, bm), lambda i: (0, i)),
            pl.BlockSpec((bm, D2), lambda i: (i, 0)),
        ],
        out_shape=[
            jax.ShapeDtypeStruct((D2, n_pad), jnp.int32),
            jax.ShapeDtypeStruct((n_pad, D2), jnp.int32),
        ],
    )(feat_pad, *wb)


def _unpack_lo(v):
    u = lax.bitcast_convert_type(v, jnp.uint32) & jnp.uint32(0xFFFF)
    return lax.bitcast_convert_type(
        lax.convert_element_type(u, jnp.uint16), jnp.bfloat16)


def _unpack_hi(v):
    u = lax.shift_right_logical(
        lax.bitcast_convert_type(v, jnp.uint32), jnp.uint32(16))
    return lax.bitcast_convert_type(
        lax.convert_element_type(u, jnp.uint16), jnp.bfloat16)


def _epilogue_body(o_ref, hd_ref, out_ref):
    xt = o_ref[...].T
    h = hd_ref[...]
    out_ref[:, :D2] = (_unpack_lo(xt).astype(jnp.float32)
                       + _unpack_lo(h).astype(jnp.float32))
    out_ref[:, D2:] = (_unpack_hi(xt).astype(jnp.float32)
                       + _unpack_hi(h).astype(jnp.float32))


def _tc_epilogue(out_sc, hdst_p, n_pad):
    bm = 512
    grid = (n_pad // bm,)
    return pl.pallas_call(
        _epilogue_body,
        grid=grid,
        in_specs=[
            pl.BlockSpec((D2, bm), lambda i: (0, i)),
            pl.BlockSpec((bm, D2), lambda i: (i, 0)),
        ],
        out_specs=pl.BlockSpec((bm, D), lambda i: (i, 0)),
        out_shape=jax.ShapeDtypeStruct((n_pad, D), jnp.float32),
    )(out_sc, hdst_p)


def _make_sc_kernel(n_pad):
    nch = n_pad // CH
    mesh = plsc.VectorSubcoreMesh(core_axis_name="c", subcore_axis_name="s")

    @functools.partial(
        pl.kernel,
        out_type=jax.ShapeDtypeStruct((D2, n_pad), jnp.int32),
        mesh=mesh,
        compiler_params=pltpu.CompilerParams(
            use_tc_tiling_on_sc=False, needs_layout_passes=False),
        scratch_types=[
            pltpu.VMEM((CPW, n_pad), jnp.int32),
            pltpu.VMEM((2, K // 2, CH), jnp.int32),
            pltpu.VMEM((2, CPW, CH), jnp.int32),
            pltpu.SemaphoreType.DMA,
            pltpu.SemaphoreType.DMA,
            pltpu.SemaphoreType.DMA,
            pltpu.SemaphoreType.DMA,
        ],
    )
    def sc_kernel(hsrc_hbm, idx_hbm, out_hbm,
                  tab_v, idx_v, out_v, isem0, isem1, osem0, osem1):
        wid = lax.axis_index("s") * NC + lax.axis_index("c")
        col0 = wid * CPW
        pltpu.sync_copy(hsrc_hbm.at[pl.ds(col0, CPW)], tab_v)
        isems = (isem0, isem1)
        osems = (osem0, osem1)

        def idx_start(ch, buf):
            pltpu.async_copy(idx_hbm.at[:, pl.ds(ch * CH, CH)],
                             idx_v.at[buf], isems[buf])

        def idx_wait(ch, buf):
            pltpu.make_async_copy(idx_hbm.at[:, pl.ds(ch * CH, CH)],
                                  idx_v.at[buf], isems[buf]).wait()

        def out_start(ch, buf):
            pltpu.async_copy(
                out_v.at[buf],
                out_hbm.at[pl.ds(col0, CPW), pl.ds(ch * CH, CH)],
                osems[buf])

        def out_wait(ch, buf):
            pltpu.make_async_copy(
                out_v.at[buf],
                out_hbm.at[pl.ds(col0, CPW), pl.ds(ch * CH, CH)],
                osems[buf]).wait()

        def chunk_compute(ch, buf):
            def group(g, carry):
                gsl = pl.ds(g * L, L)
                accs = None
                for jp in range(K // 2):
                    pair = idx_v[buf, jp, gsl]
                    iv0 = pair & jnp.int32(0xFFFF)
                    iv1 = lax.shift_right_logical(pair, jnp.int32(16))
                    for iv in (iv0, iv1):
                        if accs is None:
                            accs = [
                                plsc.bitcast(
                                    plsc.load_gather(tab_v.at[col], [iv]),
                                    jnp.bfloat16)
                                for col in range(CPW)]
                        else:
                            for col in range(CPW):
                                x = plsc.load_gather(tab_v.at[col], [iv])
                                accs[col] = jnp.maximum(
                                    accs[col], plsc.bitcast(x, jnp.bfloat16))
                for col in range(CPW):
                    out_v[buf, col, gsl] = plsc.bitcast(accs[col], jnp.int32)
                return carry

            lax.fori_loop(0, CH // L, group, 0)

        idx_start(0, 0)
        for ch in range(nch):
            buf = ch % 2
            if ch + 1 < nch:
                idx_start(ch + 1, 1 - buf)
            idx_wait(ch, buf)
            if ch >= 2:
                out_wait(ch - 2, buf)
            chunk_compute(ch, buf)
            out_start(ch, buf)
        out_wait(nch - 2, nch % 2)
        out_wait(nch - 1, (nch - 1) % 2)

    return sc_kernel


def kernel(k, src_ind, feat, W_theta, W_phi):
    n = feat.shape[0]
    n_pad = -(-n // CH) * CH
    feat_pad = jnp.pad(feat, ((0, n_pad - n), (0, 0))).astype(jnp.bfloat16)
    wt = W_theta.T
    wd = (W_phi - W_theta).T
    h_src, h_dst = _tc_matmuls(feat_pad, wt, wd, n_pad)

    idx16 = jnp.pad(src_ind.astype(jnp.uint16), ((0, n_pad - n), (0, 0)))
    idx_p = lax.bitcast_convert_type(
        idx16.reshape(n_pad, K // 2, 2), jnp.int32).T

    out_sc = _make_sc_kernel(n_pad)(h_src, idx_p)
    return _tc_epilogue(out_sc, h_dst, n_pad)[:n]


# speedup vs baseline: 1.0578x; 1.0578x over previous
import functools

import jax
import jax.numpy as jnp
from jax import lax
from jax.experimental import pallas as pl
from jax.experimental.pallas import tpu as pltpu
from jax.experimental.pallas import tpu_sc as plsc

D = 128
D2 = D // 2
K = 32
NC = 2
NS = 16
NW = NC * NS
CPW = D2 // NW
L = 16
CH = 1024


def _pack_bf16_pair(lo_f32, hi_f32):
    lo = lax.convert_element_type(
        lax.bitcast_convert_type(lo_f32.astype(jnp.bfloat16), jnp.uint16),
        jnp.uint32)
    hi = lax.convert_element_type(
        lax.bitcast_convert_type(hi_f32.astype(jnp.bfloat16), jnp.uint16),
        jnp.uint32)
    return lax.bitcast_convert_type((hi << 16) | lo, jnp.int32)


def _matmul_body(x_ref, wte_ref, wto_ref, wde_ref, wdo_ref, hs_ref, hd_ref):
    x = x_ref[...]
    hse = jnp.dot(x, wte_ref[...], preferred_element_type=jnp.float32)
    hso = jnp.dot(x, wto_ref[...], preferred_element_type=jnp.float32)
    hde = jnp.dot(x, wde_ref[...], preferred_element_type=jnp.float32)
    hdo = jnp.dot(x, wdo_ref[...], preferred_element_type=jnp.float32)
    hs_ref[...] = _pack_bf16_pair(hse, hso).T
    hd_ref[...] = _pack_bf16_pair(hde, hdo)


def _tc_matmuls(feat_pad, wt, wd, n_pad):
    bm = 512
    grid = (n_pad // bm,)
    wb = [w.astype(jnp.bfloat16)
          for w in (wt[:, :D2], wt[:, D2:], wd[:, :D2], wd[:, D2:])]
    return pl.pallas_call(
        _matmul_body,
        grid=grid,
        in_specs=[pl.BlockSpec((bm, D), lambda i: (i, 0))] + [
            pl.BlockSpec((D, D2), lambda i: (0, 0))] * 4,
        out_specs=[
            pl.BlockSpec((D2, bm), lambda i: (0, i)),
            pl.BlockSpec((bm, D2), lambda i: (i, 0)),
        ],
        out_shape=[
            jax.ShapeDtypeStruct((D2, n_pad), jnp.int32),
            jax.ShapeDtypeStruct((n_pad, D2), jnp.int32),
        ],
    )(feat_pad, *wb)


def _unpack_lo(v):
    u = lax.bitcast_convert_type(v, jnp.uint32) & jnp.uint32(0xFFFF)
    return lax.bitcast_convert_type(
        lax.convert_element_type(u, jnp.uint16), jnp.bfloat16)


def _unpack_hi(v):
    u = lax.shift_right_logical(
        lax.bitcast_convert_type(v, jnp.uint32), jnp.uint32(16))
    return lax.bitcast_convert_type(
        lax.convert_element_type(u, jnp.uint16), jnp.bfloat16)


def _epilogue_body(o_ref, hd_ref, out_ref):
    xt = o_ref[...].T
    h = hd_ref[...]
    out_ref[:, :D2] = (_unpack_lo(xt).astype(jnp.float32)
                       + _unpack_lo(h).astype(jnp.float32))
    out_ref[:, D2:] = (_unpack_hi(xt).astype(jnp.float32)
                       + _unpack_hi(h).astype(jnp.float32))


def _tc_epilogue(out_sc, hdst_p, n, n_pad):
    bm = 512
    grid = (-(-n // bm),)
    return pl.pallas_call(
        _epilogue_body,
        grid=grid,
        in_specs=[
            pl.BlockSpec((D2, bm), lambda i: (0, i)),
            pl.BlockSpec((bm, D2), lambda i: (i, 0)),
        ],
        out_specs=pl.BlockSpec((bm, D), lambda i: (i, 0)),
        out_shape=jax.ShapeDtypeStruct((n, D), jnp.float32),
    )(out_sc, hdst_p)


def _make_sc_kernel(n_pad):
    nch = n_pad // CH
    mesh = plsc.VectorSubcoreMesh(core_axis_name="c", subcore_axis_name="s")

    @functools.partial(
        pl.kernel,
        out_type=jax.ShapeDtypeStruct((D2, n_pad), jnp.int32),
        mesh=mesh,
        compiler_params=pltpu.CompilerParams(
            use_tc_tiling_on_sc=False, needs_layout_passes=False),
        scratch_types=[
            pltpu.VMEM((CPW, n_pad), jnp.int32),
            pltpu.VMEM((2, K // 2, CH), jnp.int32),
            pltpu.VMEM((2, CPW, CH), jnp.int32),
            pltpu.SemaphoreType.DMA,
            pltpu.SemaphoreType.DMA,
            pltpu.SemaphoreType.DMA,
            pltpu.SemaphoreType.DMA,
        ],
    )
    def sc_kernel(hsrc_hbm, idx_hbm, out_hbm,
                  tab_v, idx_v, out_v, isem0, isem1, osem0, osem1):
        wid = lax.axis_index("s") * NC + lax.axis_index("c")
        col0 = wid * CPW
        pltpu.sync_copy(hsrc_hbm.at[pl.ds(col0, CPW)], tab_v)
        isems = (isem0, isem1)
        osems = (osem0, osem1)

        def idx_start(ch, buf):
            pltpu.async_copy(idx_hbm.at[:, pl.ds(ch * CH, CH)],
                             idx_v.at[buf], isems[buf])

        def idx_wait(ch, buf):
            pltpu.make_async_copy(idx_hbm.at[:, pl.ds(ch * CH, CH)],
                                  idx_v.at[buf], isems[buf]).wait()

        def out_start(ch, buf):
            pltpu.async_copy(
                out_v.at[buf],
                out_hbm.at[pl.ds(col0, CPW), pl.ds(ch * CH, CH)],
                osems[buf])

        def out_wait(ch, buf):
            pltpu.make_async_copy(
                out_v.at[buf],
                out_hbm.at[pl.ds(col0, CPW), pl.ds(ch * CH, CH)],
                osems[buf]).wait()

        def chunk_compute(ch, buf):
            def group(g, carry):
                gsl = pl.ds(g * L, L)
                accs = None
                for jp in range(K // 2):
                    pair = idx_v[buf, jp, gsl]
                    iv0 = pair & jnp.int32(0xFFFF)
                    iv1 = lax.shift_right_logical(pair, jnp.int32(16))
                    for iv in (iv0, iv1):
                        if accs is None:
                            accs = [
                                plsc.bitcast(
                                    plsc.load_gather(tab_v.at[col], [iv]),
                                    jnp.bfloat16)
                                for col in range(CPW)]
                        else:
                            for col in range(CPW):
                                x = plsc.load_gather(tab_v.at[col], [iv])
                                accs[col] = jnp.maximum(
                                    accs[col], plsc.bitcast(x, jnp.bfloat16))
                for col in range(CPW):
                    out_v[buf, col, gsl] = plsc.bitcast(accs[col], jnp.int32)
                return carry

            lax.fori_loop(0, CH // L, group, 0)

        idx_start(0, 0)
        for ch in range(nch):
            buf = ch % 2
            if ch + 1 < nch:
                idx_start(ch + 1, 1 - buf)
            idx_wait(ch, buf)
            if ch >= 2:
                out_wait(ch - 2, buf)
            chunk_compute(ch, buf)
            out_start(ch, buf)
        out_wait(nch - 2, nch % 2)
        out_wait(nch - 1, (nch - 1) % 2)

    return sc_kernel


def kernel(k, src_ind, feat, W_theta, W_phi):
    n = feat.shape[0]
    n_pad = -(-n // CH) * CH
    feat_pad = jnp.pad(feat, ((0, n_pad - n), (0, 0))).astype(jnp.bfloat16)
    wt = W_theta.T
    wd = (W_phi - W_theta).T
    h_src, h_dst = _tc_matmuls(feat_pad, wt, wd, n_pad)

    idx16 = jnp.pad(src_ind.astype(jnp.uint16), ((0, n_pad - n), (0, 0)))
    idx_p = lax.bitcast_convert_type(
        idx16.reshape(n_pad, K // 2, 2), jnp.int32).T

    out_sc = _make_sc_kernel(n_pad)(h_src, idx_p)
    return _tc_epilogue(out_sc, h_dst, n, n_pad)
